# initial kernel scaffold (unmeasured)
import jax
import jax.numpy as jnp
from jax import lax
from jax.experimental import pallas as pl
from jax.experimental.pallas import tpu as pltpu


def kernel(
    x,
):
    def body(*refs):
        pass

    out_shape = jax.ShapeDtypeStruct(..., jnp.float32)
    return pl.pallas_call(body, out_shape=out_shape)(...)


# baseline (device time: 23469 ns/iter reference)
import jax
import jax.numpy as jnp
from jax import lax
from jax.experimental import pallas as pl
from jax.experimental.pallas import tpu as pltpu

N_DEV = 8
SUB = 16
LANE = 128


def kernel(x):
    m, n = x.shape
    assert m == SUB * LANE

    def body(x_ref, out_ref, comm_ref, send_sems, recv_sems):
        me = lax.axis_index("i")

        barrier_sem = pltpu.get_barrier_semaphore()
        for off in range(1, N_DEV):
            peer = (me + off) % N_DEV
            pl.semaphore_signal(
                barrier_sem, inc=1,
                device_id=(peer,), device_id_type=pl.DeviceIdType.MESH,
            )
        pl.semaphore_wait(barrier_sem, N_DEV - 1)

        x3 = x_ref[:, :].reshape(SUB, LANE, n)
        m_loc = jnp.max(x3, axis=2)
        e3 = jnp.exp(x3 - m_loc[:, :, None])
        out_ref[:, :] = e3.reshape(m, n)
        s_loc = jnp.sum(e3, axis=2)

        comm_ref[me, 0] = m_loc
        comm_ref[me, 1] = s_loc

        rdmas = []
        for off in range(1, N_DEV):
            peer = (me + off) % N_DEV
            rdma = pltpu.make_async_remote_copy(
                src_ref=comm_ref.at[me],
                dst_ref=comm_ref.at[me],
                send_sem=send_sems.at[off - 1],
                recv_sem=recv_sems.at[off - 1],
                device_id=(peer,),
                device_id_type=pl.DeviceIdType.MESH,
            )
            rdma.start()
            rdmas.append(rdma)

        for off in range(1, N_DEV):
            src = (me - off) % N_DEV
            recv = pltpu.make_async_remote_copy(
                src_ref=comm_ref.at[me],
                dst_ref=comm_ref.at[src],
                send_sem=send_sems.at[off - 1],
                recv_sem=recv_sems.at[off - 1],
                device_id=(src,),
                device_id_type=pl.DeviceIdType.MESH,
            )
            recv.wait_recv()

        stats = comm_ref[:, :, :, :]
        m_all = stats[:, 0]
        s_all = stats[:, 1]
        m_g = jnp.max(m_all, axis=0)
        s_g = jnp.sum(s_all * jnp.exp(m_all - m_g[None]), axis=0)
        scale = jnp.exp(m_loc - m_g) / s_g
        out3 = out_ref[:, :].reshape(SUB, LANE, n) * scale[:, :, None]
        out_ref[:, :] = out3.reshape(m, n)

        for rdma in rdmas:
            rdma.wait_send()

    return pl.pallas_call(
        body,
        out_shape=jax.ShapeDtypeStruct((m, n), jnp.float32),
        in_specs=[pl.BlockSpec(memory_space=pltpu.VMEM)],
        out_specs=pl.BlockSpec(memory_space=pltpu.VMEM),
        scratch_shapes=[
            pltpu.VMEM((N_DEV, 2, SUB, LANE), jnp.float32),
            pltpu.SemaphoreType.DMA((N_DEV - 1,)),
            pltpu.SemaphoreType.DMA((N_DEV - 1,)),
        ],
        compiler_params=pltpu.CompilerParams(collective_id=0),
    )(x)
